# Initial kernel scaffold; baseline (speedup 1.0000x reference)
import jax
import jax.numpy as jnp
from jax.experimental import pallas as pl


def kernel(overlap_values, overlap_row, overlap_col, edges, W1e, b1e, W2e, b2e, W1n, b1n, W2n, b2n):
    raise NotImplementedError("write your pallas kernel here")


# speedup vs baseline: 1.2265x; 1.2265x over previous
import functools

import jax
import jax.numpy as jnp
from jax import lax
from jax.experimental import pallas as pl
from jax.experimental.pallas import tpu as pltpu
from jax.experimental.pallas import tpu_sc as plsc

N = 4096
PSI = 64
NC = 2
NS = 16
LANES = 16
ROWS = 1536
NNZ_PAD = ROWS * 128
TPR = ROWS // NS
K2_TPR = ROWS // (NC * NS)
W = 1 << 20
SLAB_ROWS = 256
SLABS_PER_CORE = 8
STRIPE = W // NS
AGG_W = 4352
AGG_STRIPE = AGG_W // NS
E_EDGES = 8192

_MESH = plsc.VectorSubcoreMesh(
    core_axis_name="c", subcore_axis_name="s", num_cores=NC, num_subcores=NS)
_SC_PARAMS = pltpu.CompilerParams(use_tc_tiling_on_sc=False,
                                  needs_layout_passes=False)


def _f32(x):
    return jnp.asarray(x, jnp.float32)


def _rtne_bf16(x):
    u = lax.bitcast_convert_type(x, jnp.int32)
    u = (u + 0x7FFF + ((u >> 16) & 1)) & (-65536)
    return lax.bitcast_convert_type(u, jnp.float32)


def _k1_body(v_ref, row_ref, col_ref, w1_ref, b1_ref, w2_ref, b2_ref,
             fe_ref, lidx_ref):
    v = v_ref[...]
    acc = jnp.zeros_like(v)
    for h in range(PSI):
        hid = jnp.maximum(v * w1_ref[0, h] + b1_ref[h], 0.0)
        acc = acc + _rtne_bf16(hid) * w2_ref[h, 0]
    fe_ref[...] = acc + b2_ref[0]
    lidx_ref[...] = row_ref[...] * N + col_ref[...]


def _k1(v2, row2, col2, w1, b1, w2, b2):
    return pl.pallas_call(
        _k1_body,
        out_shape=[
            jax.ShapeDtypeStruct((ROWS, 128), jnp.float32),
            jax.ShapeDtypeStruct((ROWS, 128), jnp.int32),
        ],
        in_specs=[
            pl.BlockSpec(memory_space=pltpu.VMEM),
            pl.BlockSpec(memory_space=pltpu.VMEM),
            pl.BlockSpec(memory_space=pltpu.VMEM),
            pl.BlockSpec(memory_space=pltpu.SMEM),
            pl.BlockSpec(memory_space=pltpu.SMEM),
            pl.BlockSpec(memory_space=pltpu.SMEM),
            pl.BlockSpec(memory_space=pltpu.SMEM),
        ],
        out_specs=[
            pl.BlockSpec(memory_space=pltpu.VMEM),
            pl.BlockSpec(memory_space=pltpu.VMEM),
        ],
    )(v2, row2, col2, w1, b1, w2, b2)


def _k2_body(fe_hbm, col_hbm, out_hbm, agg_sh, zbuf, fe_t, col_t, sem):
    c = lax.axis_index("c")
    s = lax.axis_index("s")

    def zb(i, _):
        zbuf[pl.ds(i * LANES, LANES)] = jnp.zeros((LANES,), jnp.float32)
        return 0
    lax.fori_loop(0, AGG_STRIPE // LANES, zb, 0)
    pltpu.sync_copy(zbuf, agg_sh.at[pl.ds(s * AGG_STRIPE, AGG_STRIPE)])
    plsc.subcore_barrier()

    r0 = (c * NS + s) * K2_TPR
    pltpu.sync_copy(fe_hbm.at[pl.ds(r0, K2_TPR)], fe_t)
    pltpu.sync_copy(col_hbm.at[pl.ds(r0, K2_TPR)], col_t)

    def sc_group(g, _):
        hs = []
        for q in range(3):
            j = g * 3 + q
            hs.append(pltpu.async_copy(
                fe_t.at[j], agg_sh.at[col_t.at[j]], sem, add=True))
        for h in hs:
            h.wait()
        return 0
    lax.fori_loop(0, K2_TPR // 3, sc_group, 0)
    plsc.subcore_barrier()

    pltpu.sync_copy(agg_sh.at[pl.ds(s * AGG_STRIPE, AGG_STRIPE)], zbuf)
    pltpu.sync_copy(zbuf, out_hbm.at[pl.ds(c * AGG_W + s * AGG_STRIPE,
                                           AGG_STRIPE)])


_k2 = pl.kernel(
    _k2_body,
    out_type=jax.ShapeDtypeStruct((NC * AGG_W,), jnp.float32),
    mesh=_MESH,
    compiler_params=_SC_PARAMS,
    scratch_types=[
        pltpu.VMEM_SHARED((AGG_W,), jnp.float32),
        pltpu.VMEM((AGG_STRIPE,), jnp.float32),
        pltpu.VMEM((K2_TPR, 128), jnp.float32),
        pltpu.VMEM((K2_TPR, 128), jnp.int32),
        pltpu.SemaphoreType.DMA,
    ],
)


def _k3_body(agg_ref, w1_ref, b1_ref, w2_ref, b2_ref, s2_ref):
    a = agg_ref[0:N] + agg_ref[AGG_W:AGG_W + N]
    acc = jnp.zeros_like(a)
    for h in range(PSI):
        hid = jnp.maximum(a * w1_ref[0, h] + b1_ref[h], 0.0)
        acc = acc + _rtne_bf16(hid) * w2_ref[h, 0]
    st = acc + b2_ref[0]
    s2_ref[...] = st * st


def _k3(aggp, w1, b1, w2, b2):
    return pl.pallas_call(
        _k3_body,
        out_shape=jax.ShapeDtypeStruct((N,), jnp.float32),
        in_specs=[
            pl.BlockSpec(memory_space=pltpu.VMEM),
            pl.BlockSpec(memory_space=pltpu.SMEM),
            pl.BlockSpec(memory_space=pltpu.SMEM),
            pl.BlockSpec(memory_space=pltpu.SMEM),
            pl.BlockSpec(memory_space=pltpu.SMEM),
        ],
        out_specs=pl.BlockSpec(memory_space=pltpu.VMEM),
    )(aggp, w1, b1, w2, b2)


HALF = TPR // 2
QROWS = 4
QWORDS = QROWS * N


def _k4_body(lidx_hbm, v_hbm, s2_hbm, t_hbm,
             slab, zbuf, lidx_t, v_t, idxp, s2_t, rows_q, tbuf, abuf, sem):
    c = lax.axis_index("c")
    s = lax.axis_index("s")

    pltpu.sync_copy(s2_hbm, s2_t)

    def zb(i, _):
        zbuf[pl.ds(i * LANES, LANES)] = jnp.zeros((LANES,), jnp.float32)
        return 0
    lax.fori_loop(0, 4096 // LANES, zb, 0)

    lane = lax.iota(jnp.int32, LANES)

    def slab_body(si, _):
        base = (c * SLABS_PER_CORE + si) * W

        for q in range(STRIPE // 4096):
            pltpu.sync_copy(zbuf, slab.at[pl.ds(s * STRIPE + q * 4096, 4096)])
        plsc.subcore_barrier()

        def half(h, _):
            r0 = s * TPR + h * HALF
            pltpu.sync_copy(lidx_hbm.at[pl.ds(r0, HALF)], lidx_t)
            pltpu.sync_copy(v_hbm.at[pl.ds(r0, HALF)], v_t)

            def ip(j, _):
                for i in range(8):
                    vec = lidx_t[j, pl.ds(i * LANES, LANES)]
                    loc = vec - base
                    ok = (loc >= 0) & (loc < W)
                    idxp[j, pl.ds(i * LANES, LANES)] = jnp.where(ok, loc, W)
                return 0
            lax.fori_loop(0, HALF, ip, 0)

            def sc_group(g, _):
                hs = []
                for q in range(4):
                    j = g * 4 + q
                    hs.append(pltpu.async_copy(
                        v_t.at[j], slab.at[idxp.at[j]], sem, add=True))
                for hh in hs:
                    hh.wait()
                return 0
            lax.fori_loop(0, HALF // 4, sc_group, 0)
            return 0
        lax.fori_loop(0, 2, half, 0)
        plsc.subcore_barrier()

        for q in range(16 // QROWS):
            pltpu.sync_copy(
                slab.at[pl.ds(s * STRIPE + q * QWORDS, QWORDS)], rows_q)

            def red(jv, accs):
                s2v = s2_t[pl.ds(jv * LANES, LANES)]
                out = []
                for r in range(QROWS):
                    x = rows_q[pl.ds(r * N + jv * LANES, LANES)]
                    out.append(accs[r] + x * x * s2v)
                return tuple(out)
            accs = lax.fori_loop(
                0, N // LANES, red,
                tuple(jnp.zeros((LANES,), jnp.float32) for _ in range(QROWS)))
            for r in range(QROWS):
                abuf[pl.ds((q * QROWS + r) * LANES, LANES)] = accs[r]

        tvec = jnp.zeros((LANES,), jnp.float32)
        for j in range(16):
            tvec = tvec + plsc.load_gather(abuf, [lane * LANES + j])
        tbuf[...] = tvec
        pltpu.sync_copy(
            tbuf,
            t_hbm.at[pl.ds(c * (SLABS_PER_CORE * SLAB_ROWS)
                           + si * SLAB_ROWS + s * LANES, LANES)])
        plsc.subcore_barrier()
        return 0

    lax.fori_loop(0, SLABS_PER_CORE, slab_body, 0)


_k4 = pl.kernel(
    _k4_body,
    out_type=jax.ShapeDtypeStruct((N,), jnp.float32),
    mesh=_MESH,
    compiler_params=_SC_PARAMS,
    scratch_types=[
        pltpu.VMEM_SHARED((W + LANES,), jnp.float32),
        pltpu.VMEM((4096,), jnp.float32),
        pltpu.VMEM((HALF, 128), jnp.int32),
        pltpu.VMEM((HALF, 128), jnp.float32),
        pltpu.VMEM((HALF, 128), jnp.int32),
        pltpu.VMEM((N,), jnp.float32),
        pltpu.VMEM((QWORDS,), jnp.float32),
        pltpu.VMEM((LANES,), jnp.float32),
        pltpu.VMEM((LANES * LANES,), jnp.float32),
        pltpu.SemaphoreType.DMA,
    ],
)


def _k5_body(src_hbm, t_hbm, o_hbm, t_v, src_v, o_v):
    c = lax.axis_index("c")
    s = lax.axis_index("s")
    wid = c * NS + s
    chunk = E_EDGES // (NC * NS)
    pltpu.sync_copy(t_hbm, t_v)
    pltpu.sync_copy(src_hbm.at[pl.ds(wid * chunk, chunk)], src_v)

    def gat(i, _):
        idx = src_v[pl.ds(i * LANES, LANES)]
        o_v[pl.ds(i * LANES, LANES)] = plsc.load_gather(t_v, [idx])
        return 0
    lax.fori_loop(0, chunk // LANES, gat, 0)
    pltpu.sync_copy(o_v, o_hbm.at[pl.ds(wid * chunk, chunk)])


_k5 = pl.kernel(
    _k5_body,
    out_type=jax.ShapeDtypeStruct((E_EDGES,), jnp.float32),
    mesh=_MESH,
    compiler_params=_SC_PARAMS,
    scratch_types=[
        pltpu.VMEM((N,), jnp.float32),
        pltpu.VMEM((E_EDGES // (NC * NS),), jnp.int32),
        pltpu.VMEM((E_EDGES // (NC * NS),), jnp.float32),
    ],
)


def kernel(overlap_values, overlap_row, overlap_col, edges,
           W1e, b1e, W2e, b2e, W1n, b1n, W2n, b2n):
    v = _f32(overlap_values)
    row = overlap_row.astype(jnp.int32)
    col = overlap_col.astype(jnp.int32)
    src = edges[:, 0].astype(jnp.int32)
    pad = NNZ_PAD - v.shape[0]
    v2 = jnp.pad(v, (0, pad)).reshape(ROWS, 128)
    row2 = jnp.pad(row, (0, pad), constant_values=-N).reshape(ROWS, 128)
    col2 = jnp.pad(col, (0, pad), constant_values=N).reshape(ROWS, 128)

    fe2, lidx2 = _k1(v2, row2, col2, _f32(W1e), _f32(b1e),
                     _rtne_bf16(_f32(W2e)), _f32(b2e))
    aggp = _k2(fe2, col2)
    s2 = _k3(aggp, _f32(W1n), _f32(b1n), _rtne_bf16(_f32(W2n)), _f32(b2n))
    t = _k4(lidx2, v2, s2)
    o = _k5(src, t)
    return o.reshape(E_EDGES, 1)


# speedup vs baseline: 8.4913x; 6.9234x over previous
import functools

import jax
import jax.numpy as jnp
from jax import lax
from jax.experimental import pallas as pl
from jax.experimental.pallas import tpu as pltpu
from jax.experimental.pallas import tpu_sc as plsc

N = 4096
PSI = 64
NC = 2
NS = 16
LANES = 16
ROWS = 1536
NNZ_PAD = ROWS * 128
TPR = ROWS // NS
K2_TPR = ROWS // (NC * NS)
W = 1 << 20
SLAB_ROWS = 256
SLABS_PER_CORE = 8
STRIPE = W // NS
AGG_W = 4352
AGG_STRIPE = AGG_W // NS
E_EDGES = 8192

_MESH = plsc.VectorSubcoreMesh(
    core_axis_name="c", subcore_axis_name="s", num_cores=NC, num_subcores=NS)
_SC_PARAMS = pltpu.CompilerParams(use_tc_tiling_on_sc=False,
                                  needs_layout_passes=False)


def _f32(x):
    return jnp.asarray(x, jnp.float32)


def _rtne_bf16(x):
    u = lax.bitcast_convert_type(x, jnp.int32)
    u = (u + 0x7FFF + ((u >> 16) & 1)) & (-65536)
    return lax.bitcast_convert_type(u, jnp.float32)


def _k1_body(v_ref, row_ref, col_ref, w1_ref, b1_ref, w2_ref, b2_ref,
             fe_ref, lidx_ref):
    v = v_ref[...]
    acc = jnp.zeros_like(v)
    for h in range(PSI):
        hid = jnp.maximum(v * w1_ref[0, h] + b1_ref[h], 0.0)
        acc = acc + _rtne_bf16(hid) * w2_ref[h, 0]
    fe_ref[...] = acc + b2_ref[0]
    lidx_ref[...] = row_ref[...] * N + col_ref[...]


def _k1(v2, row2, col2, w1, b1, w2, b2):
    return pl.pallas_call(
        _k1_body,
        out_shape=[
            jax.ShapeDtypeStruct((ROWS, 128), jnp.float32),
            jax.ShapeDtypeStruct((ROWS, 128), jnp.int32),
        ],
        in_specs=[
            pl.BlockSpec(memory_space=pltpu.VMEM),
            pl.BlockSpec(memory_space=pltpu.VMEM),
            pl.BlockSpec(memory_space=pltpu.VMEM),
            pl.BlockSpec(memory_space=pltpu.SMEM),
            pl.BlockSpec(memory_space=pltpu.SMEM),
            pl.BlockSpec(memory_space=pltpu.SMEM),
            pl.BlockSpec(memory_space=pltpu.SMEM),
        ],
        out_specs=[
            pl.BlockSpec(memory_space=pltpu.VMEM),
            pl.BlockSpec(memory_space=pltpu.VMEM),
        ],
    )(v2, row2, col2, w1, b1, w2, b2)


def _k2_body(fe_hbm, col_hbm, out_hbm, agg_sh, zbuf, fe_t, col_t, sem):
    c = lax.axis_index("c")
    s = lax.axis_index("s")

    def zb(i, _):
        zbuf[pl.ds(i * LANES, LANES)] = jnp.zeros((LANES,), jnp.float32)
        return 0
    lax.fori_loop(0, AGG_STRIPE // LANES, zb, 0)
    pltpu.sync_copy(zbuf, agg_sh.at[pl.ds(s * AGG_STRIPE, AGG_STRIPE)])
    plsc.subcore_barrier()

    r0 = (c * NS + s) * K2_TPR
    pltpu.sync_copy(fe_hbm.at[pl.ds(r0, K2_TPR)], fe_t)
    pltpu.sync_copy(col_hbm.at[pl.ds(r0, K2_TPR)], col_t)

    def sc_group(g, _):
        hs = []
        for q in range(3):
            j = g * 3 + q
            hs.append(pltpu.async_copy(
                fe_t.at[j], agg_sh.at[col_t.at[j]], sem, add=True))
        for h in hs:
            h.wait()
        return 0
    lax.fori_loop(0, K2_TPR // 3, sc_group, 0)
    plsc.subcore_barrier()

    pltpu.sync_copy(agg_sh.at[pl.ds(s * AGG_STRIPE, AGG_STRIPE)], zbuf)
    pltpu.sync_copy(zbuf, out_hbm.at[pl.ds(c * AGG_W + s * AGG_STRIPE,
                                           AGG_STRIPE)])


_k2 = pl.kernel(
    _k2_body,
    out_type=jax.ShapeDtypeStruct((NC * AGG_W,), jnp.float32),
    mesh=_MESH,
    compiler_params=_SC_PARAMS,
    scratch_types=[
        pltpu.VMEM_SHARED((AGG_W,), jnp.float32),
        pltpu.VMEM((AGG_STRIPE,), jnp.float32),
        pltpu.VMEM((K2_TPR, 128), jnp.float32),
        pltpu.VMEM((K2_TPR, 128), jnp.int32),
        pltpu.SemaphoreType.DMA,
    ],
)


def _k3_body(agg_ref, w1_ref, b1_ref, w2_ref, b2_ref, s2_ref):
    a = agg_ref[0:N] + agg_ref[AGG_W:AGG_W + N]
    acc = jnp.zeros_like(a)
    for h in range(PSI):
        hid = jnp.maximum(a * w1_ref[0, h] + b1_ref[h], 0.0)
        acc = acc + _rtne_bf16(hid) * w2_ref[h, 0]
    st = acc + b2_ref[0]
    s2_ref[...] = st * st


def _k3(aggp, w1, b1, w2, b2):
    return pl.pallas_call(
        _k3_body,
        out_shape=jax.ShapeDtypeStruct((N,), jnp.float32),
        in_specs=[
            pl.BlockSpec(memory_space=pltpu.VMEM),
            pl.BlockSpec(memory_space=pltpu.SMEM),
            pl.BlockSpec(memory_space=pltpu.SMEM),
            pl.BlockSpec(memory_space=pltpu.SMEM),
            pl.BlockSpec(memory_space=pltpu.SMEM),
        ],
        out_specs=pl.BlockSpec(memory_space=pltpu.VMEM),
    )(aggp, w1, b1, w2, b2)


HALF = TPR // 2
QROWS = 4
QWORDS = QROWS * N


def _k4_body(lidx_hbm, v_hbm, s2_hbm, t_hbm,
             slab, zbuf, lidx_t, v_t, idxp, s2_t, rows_q, tbuf, abuf, sem):
    c = lax.axis_index("c")
    s = lax.axis_index("s")

    pltpu.sync_copy(s2_hbm, s2_t)

    def zb(i, _):
        zbuf[pl.ds(i * LANES, LANES)] = jnp.zeros((LANES,), jnp.float32)
        return 0
    lax.fori_loop(0, 4096 // LANES, zb, 0)

    lane = lax.iota(jnp.int32, LANES)

    def slab_body(si, _):
        base = (c * SLABS_PER_CORE + si) * W

        for q in range(STRIPE // 4096):
            pltpu.sync_copy(zbuf, slab.at[pl.ds(s * STRIPE + q * 4096, 4096)])
        plsc.subcore_barrier()

        def half(h, _):
            r0 = s * TPR + h * HALF
            pltpu.sync_copy(lidx_hbm.at[pl.ds(r0, HALF)], lidx_t)
            pltpu.sync_copy(v_hbm.at[pl.ds(r0, HALF)], v_t)

            def ip(j, _):
                for i in range(8):
                    vec = lidx_t[j, pl.ds(i * LANES, LANES)]
                    loc = vec - base
                    ok = (loc >= 0) & (loc < W)
                    trash = W + ((j & 127) * 8 + i) * LANES + lane
                    idxp[j, pl.ds(i * LANES, LANES)] = jnp.where(ok, loc, trash)
                return 0
            lax.fori_loop(0, HALF, ip, 0)

            def sc_group(g, _):
                hs = []
                for q in range(4):
                    j = g * 4 + q
                    hs.append(pltpu.async_copy(
                        v_t.at[j], slab.at[idxp.at[j]], sem, add=True))
                for hh in hs:
                    hh.wait()
                return 0
            lax.fori_loop(0, HALF // 4, sc_group, 0)
            return 0
        lax.fori_loop(0, 2, half, 0)
        plsc.subcore_barrier()

        for q in range(16 // QROWS):
            pltpu.sync_copy(
                slab.at[pl.ds(s * STRIPE + q * QWORDS, QWORDS)], rows_q)

            def red(jv, accs):
                s2v = s2_t[pl.ds(jv * LANES, LANES)]
                out = []
                for r in range(QROWS):
                    x = rows_q[pl.ds(r * N + jv * LANES, LANES)]
                    out.append(accs[r] + x * x * s2v)
                return tuple(out)
            accs = lax.fori_loop(
                0, N // LANES, red,
                tuple(jnp.zeros((LANES,), jnp.float32) for _ in range(QROWS)))
            for r in range(QROWS):
                abuf[pl.ds((q * QROWS + r) * LANES, LANES)] = accs[r]

        tvec = jnp.zeros((LANES,), jnp.float32)
        for j in range(16):
            tvec = tvec + plsc.load_gather(abuf, [lane * LANES + j])
        tbuf[...] = tvec
        pltpu.sync_copy(
            tbuf,
            t_hbm.at[pl.ds(c * (SLABS_PER_CORE * SLAB_ROWS)
                           + si * SLAB_ROWS + s * LANES, LANES)])
        plsc.subcore_barrier()
        return 0

    lax.fori_loop(0, SLABS_PER_CORE, slab_body, 0)


_k4 = pl.kernel(
    _k4_body,
    out_type=jax.ShapeDtypeStruct((N,), jnp.float32),
    mesh=_MESH,
    compiler_params=_SC_PARAMS,
    scratch_types=[
        pltpu.VMEM_SHARED((W + 6208,), jnp.float32),
        pltpu.VMEM((4096,), jnp.float32),
        pltpu.VMEM((HALF, 128), jnp.int32),
        pltpu.VMEM((HALF, 128), jnp.float32),
        pltpu.VMEM((HALF, 128), jnp.int32),
        pltpu.VMEM((N,), jnp.float32),
        pltpu.VMEM((QWORDS,), jnp.float32),
        pltpu.VMEM((LANES,), jnp.float32),
        pltpu.VMEM((LANES * LANES,), jnp.float32),
        pltpu.SemaphoreType.DMA,
    ],
)


def _k5_body(src_hbm, t_hbm, o_hbm, t_v, src_v, o_v):
    c = lax.axis_index("c")
    s = lax.axis_index("s")
    wid = c * NS + s
    chunk = E_EDGES // (NC * NS)
    pltpu.sync_copy(t_hbm, t_v)
    pltpu.sync_copy(src_hbm.at[pl.ds(wid * chunk, chunk)], src_v)

    def gat(i, _):
        idx = src_v[pl.ds(i * LANES, LANES)]
        o_v[pl.ds(i * LANES, LANES)] = plsc.load_gather(t_v, [idx])
        return 0
    lax.fori_loop(0, chunk // LANES, gat, 0)
    pltpu.sync_copy(o_v, o_hbm.at[pl.ds(wid * chunk, chunk)])


_k5 = pl.kernel(
    _k5_body,
    out_type=jax.ShapeDtypeStruct((E_EDGES,), jnp.float32),
    mesh=_MESH,
    compiler_params=_SC_PARAMS,
    scratch_types=[
        pltpu.VMEM((N,), jnp.float32),
        pltpu.VMEM((E_EDGES // (NC * NS),), jnp.int32),
        pltpu.VMEM((E_EDGES // (NC * NS),), jnp.float32),
    ],
)


def kernel(overlap_values, overlap_row, overlap_col, edges,
           W1e, b1e, W2e, b2e, W1n, b1n, W2n, b2n):
    v = _f32(overlap_values)
    row = overlap_row.astype(jnp.int32)
    col = overlap_col.astype(jnp.int32)
    src = edges[:, 0].astype(jnp.int32)
    pad = NNZ_PAD - v.shape[0]
    v2 = jnp.pad(v, (0, pad)).reshape(ROWS, 128)
    row2 = jnp.pad(row, (0, pad), constant_values=-N).reshape(ROWS, 128)
    col2 = jnp.pad(col, (0, pad), constant_values=N).reshape(ROWS, 128)

    fe2, lidx2 = _k1(v2, row2, col2, _f32(W1e), _f32(b1e),
                     _rtne_bf16(_f32(W2e)), _f32(b2e))
    aggp = _k2(fe2, col2)
    s2 = _k3(aggp, _f32(W1n), _f32(b1n), _rtne_bf16(_f32(W2n)), _f32(b2n))
    t = _k4(lidx2, v2, s2)
    o = _k5(src, t)
    return o.reshape(E_EDGES, 1)


# speedup vs baseline: 9.7889x; 1.1528x over previous
import functools

import jax
import jax.numpy as jnp
from jax import lax
from jax.experimental import pallas as pl
from jax.experimental.pallas import tpu as pltpu
from jax.experimental.pallas import tpu_sc as plsc

N = 4096
PSI = 64
NC = 2
NS = 16
LANES = 16
ROWS = 1536
NNZ_PAD = ROWS * 128
TPR = ROWS // NS
K2_TPR = ROWS // (NC * NS)
W = 1 << 20
SLAB_ROWS = 256
SLABS_PER_CORE = 8
STRIPE = W // NS
AGG_W = 4352
AGG_STRIPE = AGG_W // NS
E_EDGES = 8192

_MESH = plsc.VectorSubcoreMesh(
    core_axis_name="c", subcore_axis_name="s", num_cores=NC, num_subcores=NS)
_SC_PARAMS = pltpu.CompilerParams(use_tc_tiling_on_sc=False,
                                  needs_layout_passes=False)


def _f32(x):
    return jnp.asarray(x, jnp.float32)


def _rtne_bf16(x):
    u = lax.bitcast_convert_type(x, jnp.int32)
    u = (u + 0x7FFF + ((u >> 16) & 1)) & (-65536)
    return lax.bitcast_convert_type(u, jnp.float32)


def _k1_body(v_ref, row_ref, col_ref, w1_ref, b1_ref, w2_ref, b2_ref,
             fe_ref, lidx_ref):
    v = v_ref[...]
    acc = jnp.zeros_like(v)
    for h in range(PSI):
        hid = jnp.maximum(v * w1_ref[0, h] + b1_ref[h], 0.0)
        acc = acc + _rtne_bf16(hid) * w2_ref[h, 0]
    fe_ref[...] = acc + b2_ref[0]
    lidx_ref[...] = row_ref[...] * N + col_ref[...]


def _k1(v2, row2, col2, w1, b1, w2, b2):
    return pl.pallas_call(
        _k1_body,
        out_shape=[
            jax.ShapeDtypeStruct((ROWS, 128), jnp.float32),
            jax.ShapeDtypeStruct((ROWS, 128), jnp.int32),
        ],
        in_specs=[
            pl.BlockSpec(memory_space=pltpu.VMEM),
            pl.BlockSpec(memory_space=pltpu.VMEM),
            pl.BlockSpec(memory_space=pltpu.VMEM),
            pl.BlockSpec(memory_space=pltpu.SMEM),
            pl.BlockSpec(memory_space=pltpu.SMEM),
            pl.BlockSpec(memory_space=pltpu.SMEM),
            pl.BlockSpec(memory_space=pltpu.SMEM),
        ],
        out_specs=[
            pl.BlockSpec(memory_space=pltpu.VMEM),
            pl.BlockSpec(memory_space=pltpu.VMEM),
        ],
    )(v2, row2, col2, w1, b1, w2, b2)


def _k2_body(fe_hbm, col_hbm, out_hbm, agg_sh, zbuf, fe_t, col_t, sem):
    c = lax.axis_index("c")
    s = lax.axis_index("s")

    def zb(i, _):
        zbuf[pl.ds(i * LANES, LANES)] = jnp.zeros((LANES,), jnp.float32)
        return 0
    lax.fori_loop(0, AGG_STRIPE // LANES, zb, 0)
    pltpu.sync_copy(zbuf, agg_sh.at[pl.ds(s * AGG_STRIPE, AGG_STRIPE)])
    plsc.subcore_barrier()

    r0 = (c * NS + s) * K2_TPR
    pltpu.sync_copy(fe_hbm.at[pl.ds(r0, K2_TPR)], fe_t)
    pltpu.sync_copy(col_hbm.at[pl.ds(r0, K2_TPR)], col_t)

    def sc_group(g, _):
        hs = []
        for q in range(3):
            j = g * 3 + q
            hs.append(pltpu.async_copy(
                fe_t.at[j], agg_sh.at[col_t.at[j]], sem, add=True))
        for h in hs:
            h.wait()
        return 0
    lax.fori_loop(0, K2_TPR // 3, sc_group, 0)
    plsc.subcore_barrier()

    pltpu.sync_copy(agg_sh.at[pl.ds(s * AGG_STRIPE, AGG_STRIPE)], zbuf)
    pltpu.sync_copy(zbuf, out_hbm.at[pl.ds(c * AGG_W + s * AGG_STRIPE,
                                           AGG_STRIPE)])


_k2 = pl.kernel(
    _k2_body,
    out_type=jax.ShapeDtypeStruct((NC * AGG_W,), jnp.float32),
    mesh=_MESH,
    compiler_params=_SC_PARAMS,
    scratch_types=[
        pltpu.VMEM_SHARED((AGG_W,), jnp.float32),
        pltpu.VMEM((AGG_STRIPE,), jnp.float32),
        pltpu.VMEM((K2_TPR, 128), jnp.float32),
        pltpu.VMEM((K2_TPR, 128), jnp.int32),
        pltpu.SemaphoreType.DMA,
    ],
)


def _k3_body(agg_ref, w1_ref, b1_ref, w2_ref, b2_ref, s2_ref):
    a = agg_ref[0:N] + agg_ref[AGG_W:AGG_W + N]
    acc = jnp.zeros_like(a)
    for h in range(PSI):
        hid = jnp.maximum(a * w1_ref[0, h] + b1_ref[h], 0.0)
        acc = acc + _rtne_bf16(hid) * w2_ref[h, 0]
    st = acc + b2_ref[0]
    s2_ref[...] = st * st


def _k3(aggp, w1, b1, w2, b2):
    return pl.pallas_call(
        _k3_body,
        out_shape=jax.ShapeDtypeStruct((N,), jnp.float32),
        in_specs=[
            pl.BlockSpec(memory_space=pltpu.VMEM),
            pl.BlockSpec(memory_space=pltpu.SMEM),
            pl.BlockSpec(memory_space=pltpu.SMEM),
            pl.BlockSpec(memory_space=pltpu.SMEM),
            pl.BlockSpec(memory_space=pltpu.SMEM),
        ],
        out_specs=pl.BlockSpec(memory_space=pltpu.VMEM),
    )(aggp, w1, b1, w2, b2)


EIGHTH_ROWS = 2
EWORDS = EIGHTH_ROWS * N
SC_DEPTH = 8


def _k4_body(lidx_hbm, v_hbm, s2_hbm, t_hbm,
             slab, zbuf, lidx_t, v_t, idxp, s2_t, rows_db, tbuf, abuf,
             sem, semr):
    c = lax.axis_index("c")
    s = lax.axis_index("s")

    pltpu.sync_copy(s2_hbm, s2_t)

    def zb(i, _):
        zbuf[pl.ds(i * LANES, LANES)] = jnp.zeros((LANES,), jnp.float32)
        return 0
    lax.fori_loop(0, 4096 // LANES, zb, 0)

    r0 = s * TPR
    pltpu.sync_copy(lidx_hbm.at[pl.ds(r0, TPR)], lidx_t)
    pltpu.sync_copy(v_hbm.at[pl.ds(r0, TPR)], v_t)

    lane = lax.iota(jnp.int32, LANES)

    def slab_body(si, _):
        base = (c * SLABS_PER_CORE + si) * W

        zs = []
        for q in range(STRIPE // 4096):
            zs.append(pltpu.async_copy(
                zbuf, slab.at[pl.ds(s * STRIPE + q * 4096, 4096)], sem))
        for z in zs:
            z.wait()
        plsc.subcore_barrier()

        def ip(j, _):
            for i in range(8):
                vec = lidx_t[j, pl.ds(i * LANES, LANES)]
                loc = vec - base
                ok = (loc >= 0) & (loc < W)
                trash = W + (((j * 8 + i) % 384) * LANES) + lane
                idxp[j, pl.ds(i * LANES, LANES)] = jnp.where(ok, loc, trash)
            return 0
        lax.fori_loop(0, TPR, ip, 0)

        def sc_group(g, _):
            for q in range(SC_DEPTH):
                j = g * SC_DEPTH + q
                pltpu.async_copy(v_t.at[j], slab.at[idxp.at[j]], sem,
                                 add=True)

            @pl.when(g > 0)
            def _():
                for q in range(SC_DEPTH):
                    pltpu.make_async_copy(
                        v_t.at[0], slab.at[idxp.at[0]], sem).wait()
            return 0
        lax.fori_loop(0, TPR // SC_DEPTH, sc_group, 0)
        for q in range(SC_DEPTH):
            pltpu.make_async_copy(v_t.at[0], slab.at[idxp.at[0]], sem).wait()
        plsc.subcore_barrier()

        def fire(q, buf):
            return pltpu.async_copy(
                slab.at[pl.ds(s * STRIPE + q * EWORDS, EWORDS)],
                rows_db.at[buf], semr)

        fire(0, 0)
        for q in range(16 // EIGHTH_ROWS):
            pltpu.make_async_copy(slab.at[pl.ds(0, EWORDS)],
                                  rows_db.at[q % 2], semr).wait()
            if q < 7:
                fire(q + 1, (q + 1) % 2)

            def red(jv, accs):
                s2v = s2_t[pl.ds(jv * LANES, LANES)]
                out = []
                for r in range(EIGHTH_ROWS):
                    x = rows_db[q % 2, pl.ds(r * N + jv * LANES, LANES)]
                    out.append(accs[r] + x * x * s2v)
                return tuple(out)
            accs = lax.fori_loop(
                0, N // LANES, red,
                tuple(jnp.zeros((LANES,), jnp.float32)
                      for _ in range(EIGHTH_ROWS)))
            for r in range(EIGHTH_ROWS):
                abuf[pl.ds((q * EIGHTH_ROWS + r) * LANES, LANES)] = accs[r]

        tvec = jnp.zeros((LANES,), jnp.float32)
        for j in range(16):
            tvec = tvec + plsc.load_gather(abuf, [lane * LANES + j])
        tbuf[...] = tvec
        pltpu.sync_copy(
            tbuf,
            t_hbm.at[pl.ds(c * (SLABS_PER_CORE * SLAB_ROWS)
                           + si * SLAB_ROWS + s * LANES, LANES)])
        plsc.subcore_barrier()
        return 0

    lax.fori_loop(0, SLABS_PER_CORE, slab_body, 0)


_k4 = pl.kernel(
    _k4_body,
    out_type=jax.ShapeDtypeStruct((N,), jnp.float32),
    mesh=_MESH,
    compiler_params=_SC_PARAMS,
    scratch_types=[
        pltpu.VMEM_SHARED((W + 6208,), jnp.float32),
        pltpu.VMEM((4096,), jnp.float32),
        pltpu.VMEM((TPR, 128), jnp.int32),
        pltpu.VMEM((TPR, 128), jnp.float32),
        pltpu.VMEM((TPR, 128), jnp.int32),
        pltpu.VMEM((N,), jnp.float32),
        pltpu.VMEM((2, EWORDS), jnp.float32),
        pltpu.VMEM((LANES,), jnp.float32),
        pltpu.VMEM((LANES * LANES,), jnp.float32),
        pltpu.SemaphoreType.DMA,
        pltpu.SemaphoreType.DMA,
    ],
)


def _k5_body(src_hbm, t_hbm, o_hbm, t_v, src_v, o_v):
    c = lax.axis_index("c")
    s = lax.axis_index("s")
    wid = c * NS + s
    chunk = E_EDGES // (NC * NS)
    pltpu.sync_copy(t_hbm, t_v)
    pltpu.sync_copy(src_hbm.at[pl.ds(wid * chunk, chunk)], src_v)

    def gat(i, _):
        idx = src_v[pl.ds(i * LANES, LANES)]
        o_v[pl.ds(i * LANES, LANES)] = plsc.load_gather(t_v, [idx])
        return 0
    lax.fori_loop(0, chunk // LANES, gat, 0)
    pltpu.sync_copy(o_v, o_hbm.at[pl.ds(wid * chunk, chunk)])


_k5 = pl.kernel(
    _k5_body,
    out_type=jax.ShapeDtypeStruct((E_EDGES,), jnp.float32),
    mesh=_MESH,
    compiler_params=_SC_PARAMS,
    scratch_types=[
        pltpu.VMEM((N,), jnp.float32),
        pltpu.VMEM((E_EDGES // (NC * NS),), jnp.int32),
        pltpu.VMEM((E_EDGES // (NC * NS),), jnp.float32),
    ],
)


def kernel(overlap_values, overlap_row, overlap_col, edges,
           W1e, b1e, W2e, b2e, W1n, b1n, W2n, b2n):
    v = _f32(overlap_values)
    row = overlap_row.astype(jnp.int32)
    col = overlap_col.astype(jnp.int32)
    src = edges[:, 0].astype(jnp.int32)
    pad = NNZ_PAD - v.shape[0]
    v2 = jnp.pad(v, (0, pad)).reshape(ROWS, 128)
    row2 = jnp.pad(row, (0, pad), constant_values=-N).reshape(ROWS, 128)
    col2 = jnp.pad(col, (0, pad), constant_values=N).reshape(ROWS, 128)

    fe2, lidx2 = _k1(v2, row2, col2, _f32(W1e), _f32(b1e),
                     _rtne_bf16(_f32(W2e)), _f32(b2e))
    aggp = _k2(fe2, col2)
    s2 = _k3(aggp, _f32(W1n), _f32(b1n), _rtne_bf16(_f32(W2n)), _f32(b2n))
    t = _k4(lidx2, v2, s2)
    o = _k5(src, t)
    return o.reshape(E_EDGES, 1)


# speedup vs baseline: 10.9531x; 1.1189x over previous
import functools

import jax
import jax.numpy as jnp
from jax import lax
from jax.experimental import pallas as pl
from jax.experimental.pallas import tpu as pltpu
from jax.experimental.pallas import tpu_sc as plsc

N = 4096
PSI = 64
NC = 2
NS = 16
LANES = 16
ROWS = 1536
NNZ_PAD = ROWS * 128
TPR = ROWS // NS
K2_TPR = ROWS // (NC * NS)
W = 1 << 20
SLAB_ROWS = 256
SLABS_PER_CORE = 8
STRIPE = W // NS
AGG_W = 4352
AGG_STRIPE = AGG_W // NS
E_EDGES = 8192

_MESH = plsc.VectorSubcoreMesh(
    core_axis_name="c", subcore_axis_name="s", num_cores=NC, num_subcores=NS)
_SC_PARAMS = pltpu.CompilerParams(use_tc_tiling_on_sc=False,
                                  needs_layout_passes=False)


def _f32(x):
    return jnp.asarray(x, jnp.float32)


def _rtne_bf16(x):
    u = lax.bitcast_convert_type(x, jnp.int32)
    u = (u + 0x7FFF + ((u >> 16) & 1)) & (-65536)
    return lax.bitcast_convert_type(u, jnp.float32)


def _k1_body(v_ref, row_ref, col_ref, w1_ref, b1_ref, w2_ref, b2_ref,
             fe_ref, lidx_ref):
    v = v_ref[...]
    acc = jnp.zeros_like(v)
    for h in range(PSI):
        hid = jnp.maximum(v * w1_ref[0, h] + b1_ref[h], 0.0)
        acc = acc + _rtne_bf16(hid) * w2_ref[h, 0]
    fe_ref[...] = acc + b2_ref[0]
    lidx_ref[...] = row_ref[...] * N + col_ref[...]


def _k1(v2, row2, col2, w1, b1, w2, b2):
    return pl.pallas_call(
        _k1_body,
        out_shape=[
            jax.ShapeDtypeStruct((ROWS, 128), jnp.float32),
            jax.ShapeDtypeStruct((ROWS, 128), jnp.int32),
        ],
        in_specs=[
            pl.BlockSpec(memory_space=pltpu.VMEM),
            pl.BlockSpec(memory_space=pltpu.VMEM),
            pl.BlockSpec(memory_space=pltpu.VMEM),
            pl.BlockSpec(memory_space=pltpu.SMEM),
            pl.BlockSpec(memory_space=pltpu.SMEM),
            pl.BlockSpec(memory_space=pltpu.SMEM),
            pl.BlockSpec(memory_space=pltpu.SMEM),
        ],
        out_specs=[
            pl.BlockSpec(memory_space=pltpu.VMEM),
            pl.BlockSpec(memory_space=pltpu.VMEM),
        ],
    )(v2, row2, col2, w1, b1, w2, b2)


def _k2_body(fe_hbm, col_hbm, out_hbm, agg_sh, zbuf, fe_t, col_t, sem):
    c = lax.axis_index("c")
    s = lax.axis_index("s")

    def zb(i, _):
        zbuf[pl.ds(i * LANES, LANES)] = jnp.zeros((LANES,), jnp.float32)
        return 0
    lax.fori_loop(0, AGG_STRIPE // LANES, zb, 0)
    pltpu.sync_copy(zbuf, agg_sh.at[pl.ds(s * AGG_STRIPE, AGG_STRIPE)])
    plsc.subcore_barrier()

    r0 = (c * NS + s) * K2_TPR
    pltpu.sync_copy(fe_hbm.at[pl.ds(r0, K2_TPR)], fe_t)
    pltpu.sync_copy(col_hbm.at[pl.ds(r0, K2_TPR)], col_t)

    def sc_group(g, _):
        for q in range(8):
            j = g * 8 + q
            pltpu.async_copy(fe_t.at[j], agg_sh.at[col_t.at[j]], sem,
                             add=True)

        @pl.when(g > 0)
        def _():
            for q in range(8):
                pltpu.make_async_copy(
                    fe_t.at[0], agg_sh.at[col_t.at[0]], sem).wait()
        return 0
    lax.fori_loop(0, K2_TPR // 8, sc_group, 0)
    for q in range(8):
        pltpu.make_async_copy(fe_t.at[0], agg_sh.at[col_t.at[0]], sem).wait()
    plsc.subcore_barrier()

    pltpu.sync_copy(agg_sh.at[pl.ds(s * AGG_STRIPE, AGG_STRIPE)], zbuf)
    pltpu.sync_copy(zbuf, out_hbm.at[pl.ds(c * AGG_W + s * AGG_STRIPE,
                                           AGG_STRIPE)])


_k2 = pl.kernel(
    _k2_body,
    out_type=jax.ShapeDtypeStruct((NC * AGG_W,), jnp.float32),
    mesh=_MESH,
    compiler_params=_SC_PARAMS,
    scratch_types=[
        pltpu.VMEM_SHARED((AGG_W,), jnp.float32),
        pltpu.VMEM((AGG_STRIPE,), jnp.float32),
        pltpu.VMEM((K2_TPR, 128), jnp.float32),
        pltpu.VMEM((K2_TPR, 128), jnp.int32),
        pltpu.SemaphoreType.DMA,
    ],
)


def _k3_body(agg_ref, w1_ref, b1_ref, w2_ref, b2_ref, s2_ref):
    a = agg_ref[0:N] + agg_ref[AGG_W:AGG_W + N]
    acc = jnp.zeros_like(a)
    for h in range(PSI):
        hid = jnp.maximum(a * w1_ref[0, h] + b1_ref[h], 0.0)
        acc = acc + _rtne_bf16(hid) * w2_ref[h, 0]
    st = acc + b2_ref[0]
    s2_ref[...] = st * st


def _k3(aggp, w1, b1, w2, b2):
    return pl.pallas_call(
        _k3_body,
        out_shape=jax.ShapeDtypeStruct((N,), jnp.float32),
        in_specs=[
            pl.BlockSpec(memory_space=pltpu.VMEM),
            pl.BlockSpec(memory_space=pltpu.SMEM),
            pl.BlockSpec(memory_space=pltpu.SMEM),
            pl.BlockSpec(memory_space=pltpu.SMEM),
            pl.BlockSpec(memory_space=pltpu.SMEM),
        ],
        out_specs=pl.BlockSpec(memory_space=pltpu.VMEM),
    )(aggp, w1, b1, w2, b2)


EIGHTH_ROWS = 2
EWORDS = EIGHTH_ROWS * N
SC_DEPTH = 8


def _k4_body(lidx_hbm, v_hbm, s2_hbm, t_hbm,
             slab, zbuf, lidx_t, v_t, idxp, s2_t, rows_db, tbuf, abuf,
             sem, semr):
    c = lax.axis_index("c")
    s = lax.axis_index("s")

    pltpu.sync_copy(s2_hbm, s2_t)

    def zb(i, _):
        zbuf[pl.ds(i * LANES, LANES)] = jnp.zeros((LANES,), jnp.float32)
        return 0
    lax.fori_loop(0, 4096 // LANES, zb, 0)

    r0 = s * TPR
    pltpu.sync_copy(lidx_hbm.at[pl.ds(r0, TPR)], lidx_t)
    pltpu.sync_copy(v_hbm.at[pl.ds(r0, TPR)], v_t)

    lane = lax.iota(jnp.int32, LANES)

    def slab_body(si, _):
        base = (c * SLABS_PER_CORE + si) * W

        zs = []
        for q in range(STRIPE // 4096):
            zs.append(pltpu.async_copy(
                zbuf, slab.at[pl.ds(s * STRIPE + q * 4096, 4096)], sem))
        for z in zs:
            z.wait()
        plsc.subcore_barrier()

        def ip(j, _):
            for i in range(8):
                vec = lidx_t[j, pl.ds(i * LANES, LANES)]
                loc = vec - base
                ok = (loc >= 0) & (loc < W)
                trash = W + (((j * 8 + i) % 384) * LANES) + lane
                idxp[j, pl.ds(i * LANES, LANES)] = jnp.where(ok, loc, trash)
            return 0
        lax.fori_loop(0, TPR, ip, 0)

        def sc_group(g, _):
            for q in range(SC_DEPTH):
                j = g * SC_DEPTH + q
                pltpu.async_copy(v_t.at[j], slab.at[idxp.at[j]], sem,
                                 add=True)

            @pl.when(g > 0)
            def _():
                for q in range(SC_DEPTH):
                    pltpu.make_async_copy(
                        v_t.at[0], slab.at[idxp.at[0]], sem).wait()
            return 0
        lax.fori_loop(0, TPR // SC_DEPTH, sc_group, 0)
        for q in range(SC_DEPTH):
            pltpu.make_async_copy(v_t.at[0], slab.at[idxp.at[0]], sem).wait()
        plsc.subcore_barrier()

        def fire(q, buf):
            return pltpu.async_copy(
                slab.at[pl.ds(s * STRIPE + q * EWORDS, EWORDS)],
                rows_db.at[buf], semr)

        fire(0, 0)
        for q in range(16 // EIGHTH_ROWS):
            pltpu.make_async_copy(slab.at[pl.ds(0, EWORDS)],
                                  rows_db.at[q % 2], semr).wait()
            if q < 7:
                fire(q + 1, (q + 1) % 2)

            def red(i, accs):
                out = list(accs)
                for u in range(4):
                    jv = i * 4 + u
                    s2v = s2_t[pl.ds(jv * LANES, LANES)]
                    for r in range(EIGHTH_ROWS):
                        x = rows_db[q % 2, pl.ds(r * N + jv * LANES, LANES)]
                        out[r] = out[r] + x * x * s2v
                return tuple(out)
            accs = lax.fori_loop(
                0, N // LANES // 4, red,
                tuple(jnp.zeros((LANES,), jnp.float32)
                      for _ in range(EIGHTH_ROWS)))
            for r in range(EIGHTH_ROWS):
                abuf[pl.ds((q * EIGHTH_ROWS + r) * LANES, LANES)] = accs[r]

        tvec = jnp.zeros((LANES,), jnp.float32)
        for j in range(16):
            tvec = tvec + plsc.load_gather(abuf, [lane * LANES + j])
        tbuf[...] = tvec
        pltpu.sync_copy(
            tbuf,
            t_hbm.at[pl.ds(c * (SLABS_PER_CORE * SLAB_ROWS)
                           + si * SLAB_ROWS + s * LANES, LANES)])
        plsc.subcore_barrier()
        return 0

    lax.fori_loop(0, SLABS_PER_CORE, slab_body, 0)


_k4 = pl.kernel(
    _k4_body,
    out_type=jax.ShapeDtypeStruct((N,), jnp.float32),
    mesh=_MESH,
    compiler_params=_SC_PARAMS,
    scratch_types=[
        pltpu.VMEM_SHARED((W + 6208,), jnp.float32),
        pltpu.VMEM((4096,), jnp.float32),
        pltpu.VMEM((TPR, 128), jnp.int32),
        pltpu.VMEM((TPR, 128), jnp.float32),
        pltpu.VMEM((TPR, 128), jnp.int32),
        pltpu.VMEM((N,), jnp.float32),
        pltpu.VMEM((2, EWORDS), jnp.float32),
        pltpu.VMEM((LANES,), jnp.float32),
        pltpu.VMEM((LANES * LANES,), jnp.float32),
        pltpu.SemaphoreType.DMA,
        pltpu.SemaphoreType.DMA,
    ],
)


def _k5_body(src_hbm, t_hbm, o_hbm, t_v, src_v, o_v):
    c = lax.axis_index("c")
    s = lax.axis_index("s")
    wid = c * NS + s
    chunk = E_EDGES // (NC * NS)
    pltpu.sync_copy(t_hbm, t_v)
    pltpu.sync_copy(src_hbm.at[pl.ds(wid * chunk, chunk)], src_v)

    def gat(i, _):
        idx = src_v[pl.ds(i * LANES, LANES)]
        o_v[pl.ds(i * LANES, LANES)] = plsc.load_gather(t_v, [idx])
        return 0
    lax.fori_loop(0, chunk // LANES, gat, 0)
    pltpu.sync_copy(o_v, o_hbm.at[pl.ds(wid * chunk, chunk)])


_k5 = pl.kernel(
    _k5_body,
    out_type=jax.ShapeDtypeStruct((E_EDGES,), jnp.float32),
    mesh=_MESH,
    compiler_params=_SC_PARAMS,
    scratch_types=[
        pltpu.VMEM((N,), jnp.float32),
        pltpu.VMEM((E_EDGES // (NC * NS),), jnp.int32),
        pltpu.VMEM((E_EDGES // (NC * NS),), jnp.float32),
    ],
)


def kernel(overlap_values, overlap_row, overlap_col, edges,
           W1e, b1e, W2e, b2e, W1n, b1n, W2n, b2n):
    v = _f32(overlap_values)
    row = overlap_row.astype(jnp.int32)
    col = overlap_col.astype(jnp.int32)
    src = edges[:, 0].astype(jnp.int32)
    pad = NNZ_PAD - v.shape[0]
    v2 = jnp.pad(v, (0, pad)).reshape(ROWS, 128)
    row2 = jnp.pad(row, (0, pad), constant_values=-N).reshape(ROWS, 128)
    col2 = jnp.pad(col, (0, pad), constant_values=N).reshape(ROWS, 128)

    fe2, lidx2 = _k1(v2, row2, col2, _f32(W1e), _f32(b1e),
                     _rtne_bf16(_f32(W2e)), _f32(b2e))
    aggp = _k2(fe2, col2)
    s2 = _k3(aggp, _f32(W1n), _f32(b1n), _rtne_bf16(_f32(W2n)), _f32(b2n))
    t = _k4(lidx2, v2, s2)
    o = _k5(src, t)
    return o.reshape(E_EDGES, 1)
